# baseline (speedup 1.0000x reference)
import functools

import jax
import jax.numpy as jnp
from jax import lax
from jax.experimental import pallas as pl
from jax.experimental.pallas import tpu as pltpu
from jax.experimental.pallas import tpu_sc as plsc

N_NODES = 10000
N_EDGES = 320000
NPAD = 10240
EPAD = 327680
CH = 16
D = 128
NC = 2
NS = 16
NW = NC * NS
ECHUNK = 128
CHUNKS_PER_W = EPAD // NW // ECHUNK
C_CORE0 = 40
C_CORE1 = 120
C_MAX = max(C_CORE0, C_CORE1)
COLS_PER_TILE = NPAD // NS


def _deg_kernel_body(dst_hbm, w_hbm, out_hbm, dst_scr, w_scr, zbuf, acc_sh):
    cid = lax.axis_index("c")
    sid = lax.axis_index("s")
    wid = sid * NC + cid

    def zb(i, _):
        zbuf[pl.ds(i * 16, 16)] = jnp.zeros((16,), jnp.float32)
        return 0
    lax.fori_loop(0, ECHUNK // 16, zb, 0)
    for k in range(COLS_PER_TILE // ECHUNK):
        pltpu.sync_copy(zbuf, acc_sh.at[pl.ds(sid * COLS_PER_TILE + k * ECHUNK, ECHUNK)])
    plsc.subcore_barrier()

    base = wid * CHUNKS_PER_W
    pltpu.sync_copy(dst_hbm.at[pl.ds(base, CHUNKS_PER_W)], dst_scr)
    pltpu.sync_copy(w_hbm.at[pl.ds(base, CHUNKS_PER_W)], w_scr)

    def body(j, _):
        pltpu.sync_copy(w_scr.at[j], acc_sh.at[dst_scr.at[j]], add=True)
        return 0
    lax.fori_loop(0, CHUNKS_PER_W, body, 0)
    plsc.subcore_barrier()

    for k in range(COLS_PER_TILE // ECHUNK):
        sl = pl.ds(sid * COLS_PER_TILE + k * ECHUNK, ECHUNK)
        pltpu.sync_copy(acc_sh.at[sl], out_hbm.at[cid, sl])


NHALF = NPAD // 2
ROWS_PER_PASS_TILE = NHALF // NS


def _prop_kernel_body(y_hbm, src_hbm, dst_hbm, w_hbm, out_hbm,
                      src_scr, dst_scr, w_scr, dst2_scr, wm_scr,
                      rows_a, rows_b, acc_sh, semg_a, semg_b):
    cid = lax.axis_index("c")
    sid = lax.axis_index("s")
    wid = sid * NC + cid

    nch = jnp.where(cid == 0, C_CORE0, C_CORE1)
    base = jnp.where(cid == 0, sid * C_CORE0,
                     NS * C_CORE0 + sid * C_CORE1)
    pltpu.sync_copy(src_hbm.at[pl.ds(base, C_MAX)], src_scr)
    pltpu.sync_copy(dst_hbm.at[pl.ds(base, C_MAX)], dst_scr)
    pltpu.sync_copy(w_hbm.at[pl.ds(base, C_MAX)], w_scr)

    bufs = ((rows_a, semg_a), (rows_b, semg_b))
    for p in range(2):
        lo = p * NHALF
        tb = sid * ROWS_PER_PASS_TILE

        def zb(i, _):
            for k in range(D // 16):
                rows_a[i, pl.ds(k * 16, 16)] = jnp.zeros((16,), jnp.float32)
            return 0
        lax.fori_loop(0, ECHUNK, zb, 0)
        pltpu.sync_copy(rows_a, acc_sh.at[pl.ds(tb, ECHUNK)])
        pltpu.sync_copy(rows_a, acc_sh.at[pl.ds(tb + ECHUNK, ECHUNK)])
        pltpu.sync_copy(rows_a.at[pl.ds(0, 64)], acc_sh.at[pl.ds(tb + 2 * ECHUNK, 64)])
        plsc.subcore_barrier()

        pltpu.async_copy(y_hbm.at[src_scr.at[0]], rows_a, semg_a)

        def body(i, _):
            for b in range(2):
                rows, semg = bufs[b]
                rows_o, semg_o = bufs[1 - b]
                j = i * 2 + b
                @pl.when(j + 1 < nch)
                def _():
                    pltpu.async_copy(y_hbm.at[src_scr.at[j + 1]], rows_o, semg_o)
                for g in range(ECHUNK // 16):
                    gsl = pl.ds(g * 16, 16)
                    dv = dst_scr[j, gsl]
                    inr = (dv >= lo) & (dv < lo + NHALF)
                    wm_scr[gsl] = jnp.where(inr, w_scr[j, gsl], 0.0)
                    dst2_scr[gsl] = jnp.clip(dv - lo, 0, NHALF - 1)
                pltpu.make_async_copy(y_hbm.at[src_scr.at[j]], rows, semg).wait()
                for g in range(ECHUNK // 16):
                    wmask = wm_scr[pl.ds(g * 16, 16)]
                    for l in range(16):
                        e = g * 16 + l
                        rows[e, 0:16] = rows[e, 0:16] * wmask[l]
                pltpu.sync_copy(rows, acc_sh.at[dst2_scr], add=True)
            return 0
        lax.fori_loop(0, nch // 2, body, 0)
        plsc.subcore_barrier()

        pltpu.sync_copy(acc_sh.at[pl.ds(tb, ECHUNK)],
                        out_hbm.at[cid, pl.ds(lo + tb, ECHUNK)])
        pltpu.sync_copy(acc_sh.at[pl.ds(tb + ECHUNK, ECHUNK)],
                        out_hbm.at[cid, pl.ds(lo + tb + ECHUNK, ECHUNK)])
        pltpu.sync_copy(acc_sh.at[pl.ds(tb + 2 * ECHUNK, 64)],
                        out_hbm.at[cid, pl.ds(lo + tb + 2 * ECHUNK, 64)])
        plsc.subcore_barrier()


@functools.lru_cache(maxsize=None)
def _sc_kernels():
    mesh = plsc.VectorSubcoreMesh(core_axis_name="c", subcore_axis_name="s",
                                  num_cores=NC, num_subcores=NS)
    deg = pl.kernel(
        _deg_kernel_body,
        out_type=jax.ShapeDtypeStruct((NC, NPAD), jnp.float32),
        mesh=mesh,
        scratch_types=[
            pltpu.VMEM((CHUNKS_PER_W, ECHUNK), jnp.int32),
            pltpu.VMEM((CHUNKS_PER_W, ECHUNK), jnp.float32),
            pltpu.VMEM((ECHUNK,), jnp.float32),
            pltpu.VMEM_SHARED((NPAD,), jnp.float32),
        ],
    )
    prop = pl.kernel(
        _prop_kernel_body,
        out_type=jax.ShapeDtypeStruct((NC, NPAD, D), jnp.float32),
        mesh=mesh,
        scratch_types=[
            pltpu.VMEM((C_MAX, ECHUNK), jnp.int32),
            pltpu.VMEM((C_MAX, ECHUNK), jnp.int32),
            pltpu.VMEM((C_MAX, ECHUNK), jnp.float32),
            pltpu.VMEM((ECHUNK,), jnp.int32),
            pltpu.VMEM((ECHUNK,), jnp.float32),
            pltpu.VMEM((ECHUNK, D), jnp.float32),
            pltpu.VMEM((ECHUNK, D), jnp.float32),
            pltpu.VMEM_SHARED((NHALF, D), jnp.float32),
            pltpu.SemaphoreType.DMA,
            pltpu.SemaphoreType.DMA,
        ],
    )
    return deg, prop


def _stage0_body(degp_ref, z_ref, dis_ref, y1_ref):
    degp = degp_ref[...]
    ones = jnp.ones((NC, 1), jnp.float32)
    deg = lax.dot_general(degp, ones, (((0,), (0,)), ((), ())),
                          preferred_element_type=jnp.float32) + 1.0
    dis = jnp.where(deg > 0, lax.rsqrt(deg), 0.0)
    dis_ref[...] = dis
    y1_ref[...] = dis * z_ref[...]


_stage0 = pl.pallas_call(
    _stage0_body,
    out_shape=(jax.ShapeDtypeStruct((NPAD, 1), jnp.float32),
               jax.ShapeDtypeStruct((NPAD, D), jnp.float32)),
)


def _stageL_body(accp_ref, y_ref, dis_ref, w_ref, b_ref, out_ref):
    dis = dis_ref[...]
    s = dis * (accp_ref[0] + accp_ref[1] + y_ref[...])
    h = lax.dot_general(s, w_ref[...], (((1,), (0,)), ((), ())),
                        preferred_element_type=jnp.float32) + b_ref[...]
    out_ref[...] = dis * jnp.maximum(h, 0.0)


_stageL = pl.pallas_call(
    _stageL_body,
    out_shape=jax.ShapeDtypeStruct((NPAD, D), jnp.float32),
)


def _stage3_body(accp_ref, y_ref, dis_ref, w_ref, b_ref, out_ref):
    s = dis_ref[...] * (accp_ref[0] + accp_ref[1] + y_ref[...])
    out_ref[...] = lax.dot_general(s, w_ref[...], (((1,), (0,)), ((), ())),
                                   preferred_element_type=jnp.float32) + b_ref[...]


_stage3 = pl.pallas_call(
    _stage3_body,
    out_shape=jax.ShapeDtypeStruct((NPAD, 128), jnp.float32),
)


def kernel(z, edge_index, edge_weight, W1, b1, W2, b2, W3, b3):
    src = edge_index[0].astype(jnp.int32)
    dst = edge_index[1].astype(jnp.int32)
    epad = EPAD - N_EDGES
    src_r = jnp.concatenate([src, jnp.zeros((epad,), jnp.int32)]).reshape(-1, ECHUNK)
    dst_r = jnp.concatenate([dst, jnp.zeros((epad,), jnp.int32)]).reshape(-1, ECHUNK)
    w_r = jnp.concatenate(
        [edge_weight, jnp.zeros((epad,), jnp.float32)]).reshape(-1, ECHUNK)
    z2d = jnp.zeros((NPAD, D), jnp.float32).at[:N_NODES, :2].set(z)

    W1p = jnp.zeros((D, D), jnp.float32).at[:2, :8].set(W1)
    b1p = jnp.zeros((1, D), jnp.float32).at[0, :8].set(b1)
    W2p = jnp.zeros((D, D), jnp.float32).at[:8, :CH].set(W2)
    b2p = jnp.zeros((1, D), jnp.float32).at[0, :CH].set(b2)
    W3p = jnp.zeros((D, 128), jnp.float32).at[:CH, :].set(W3)
    b3p = b3.reshape(1, 128)

    deg_kernel, prop_kernel = _sc_kernels()
    degp = deg_kernel(dst_r, w_r)
    dis, y1 = _stage0(degp, z2d)
    acc1 = prop_kernel(y1, src_r, dst_r, w_r)
    y2 = _stageL(acc1, y1, dis, W1p, b1p)
    acc2 = prop_kernel(y2, src_r, dst_r, w_r)
    y3 = _stageL(acc2, y2, dis, W2p, b2p)
    acc3 = prop_kernel(y3, src_r, dst_r, w_r)
    out = _stage3(acc3, y3, dis, W3p, b3p)
    return out[:N_NODES]


# speedup vs baseline: 1.1747x; 1.1747x over previous
import functools

import jax
import jax.numpy as jnp
from jax import lax
from jax.experimental import pallas as pl
from jax.experimental.pallas import tpu as pltpu
from jax.experimental.pallas import tpu_sc as plsc

N_NODES = 10000
N_EDGES = 320000
NPAD = 10240
EPAD = 327680
CH = 16
D = 128
NC = 2
NS = 16
NW = NC * NS
ECHUNK = 128
CHUNKS_PER_W = EPAD // NW // ECHUNK
C_CORE0 = 120
C_CORE1 = 40
C_MAX = max(C_CORE0, C_CORE1)
COLS_PER_TILE = NPAD // NS


def _deg_kernel_body(dst_hbm, w_hbm, out_hbm, dst_scr, w_scr, zbuf, acc_sh):
    cid = lax.axis_index("c")
    sid = lax.axis_index("s")
    wid = sid * NC + cid

    def zb(i, _):
        zbuf[pl.ds(i * 16, 16)] = jnp.zeros((16,), jnp.float32)
        return 0
    lax.fori_loop(0, ECHUNK // 16, zb, 0)
    for k in range(COLS_PER_TILE // ECHUNK):
        pltpu.sync_copy(zbuf, acc_sh.at[pl.ds(sid * COLS_PER_TILE + k * ECHUNK, ECHUNK)])
    plsc.subcore_barrier()

    base = wid * CHUNKS_PER_W
    pltpu.sync_copy(dst_hbm.at[pl.ds(base, CHUNKS_PER_W)], dst_scr)
    pltpu.sync_copy(w_hbm.at[pl.ds(base, CHUNKS_PER_W)], w_scr)

    def body(j, _):
        pltpu.sync_copy(w_scr.at[j], acc_sh.at[dst_scr.at[j]], add=True)
        return 0
    lax.fori_loop(0, CHUNKS_PER_W, body, 0)
    plsc.subcore_barrier()

    for k in range(COLS_PER_TILE // ECHUNK):
        sl = pl.ds(sid * COLS_PER_TILE + k * ECHUNK, ECHUNK)
        pltpu.sync_copy(acc_sh.at[sl], out_hbm.at[cid, sl])


NHALF = NPAD // 2
ROWS_PER_PASS_TILE = NHALF // NS


def _prop_kernel_body(y_hbm, src_hbm, dst_hbm, w_hbm, out_hbm,
                      src_scr, dst_scr, w_scr, dst2_scr, wm_scr,
                      rows_a, rows_b, acc_sh, semg_a, semg_b):
    cid = lax.axis_index("c")
    sid = lax.axis_index("s")
    wid = sid * NC + cid

    nch = jnp.where(cid == 0, C_CORE0, C_CORE1)
    base = jnp.where(cid == 0, sid * C_CORE0,
                     NS * C_CORE0 + sid * C_CORE1)
    pltpu.sync_copy(src_hbm.at[pl.ds(base, C_MAX)], src_scr)
    pltpu.sync_copy(dst_hbm.at[pl.ds(base, C_MAX)], dst_scr)
    pltpu.sync_copy(w_hbm.at[pl.ds(base, C_MAX)], w_scr)

    bufs = ((rows_a, semg_a), (rows_b, semg_b))
    for p in range(2):
        lo = p * NHALF
        tb = sid * ROWS_PER_PASS_TILE

        def zb(i, _):
            for k in range(D // 16):
                rows_a[i, pl.ds(k * 16, 16)] = jnp.zeros((16,), jnp.float32)
            return 0
        lax.fori_loop(0, ECHUNK, zb, 0)
        pltpu.sync_copy(rows_a, acc_sh.at[pl.ds(tb, ECHUNK)])
        pltpu.sync_copy(rows_a, acc_sh.at[pl.ds(tb + ECHUNK, ECHUNK)])
        pltpu.sync_copy(rows_a.at[pl.ds(0, 64)], acc_sh.at[pl.ds(tb + 2 * ECHUNK, 64)])
        plsc.subcore_barrier()

        pltpu.async_copy(y_hbm.at[src_scr.at[0]], rows_a, semg_a)

        def body(i, _):
            for b in range(2):
                rows, semg = bufs[b]
                rows_o, semg_o = bufs[1 - b]
                j = i * 2 + b
                @pl.when(j + 1 < nch)
                def _():
                    pltpu.async_copy(y_hbm.at[src_scr.at[j + 1]], rows_o, semg_o)
                for g in range(ECHUNK // 16):
                    gsl = pl.ds(g * 16, 16)
                    dv = dst_scr[j, gsl]
                    inr = (dv >= lo) & (dv < lo + NHALF)
                    wm_scr[gsl] = jnp.where(inr, w_scr[j, gsl], 0.0)
                    dst2_scr[gsl] = jnp.clip(dv - lo, 0, NHALF - 1)
                pltpu.make_async_copy(y_hbm.at[src_scr.at[j]], rows, semg).wait()
                for g in range(ECHUNK // 16):
                    wmask = wm_scr[pl.ds(g * 16, 16)]
                    for l in range(16):
                        e = g * 16 + l
                        rows[e, 0:16] = rows[e, 0:16] * wmask[l]
                pltpu.sync_copy(rows, acc_sh.at[dst2_scr], add=True)
            return 0
        lax.fori_loop(0, nch // 2, body, 0)
        plsc.subcore_barrier()

        pltpu.sync_copy(acc_sh.at[pl.ds(tb, ECHUNK)],
                        out_hbm.at[cid, pl.ds(lo + tb, ECHUNK)])
        pltpu.sync_copy(acc_sh.at[pl.ds(tb + ECHUNK, ECHUNK)],
                        out_hbm.at[cid, pl.ds(lo + tb + ECHUNK, ECHUNK)])
        pltpu.sync_copy(acc_sh.at[pl.ds(tb + 2 * ECHUNK, 64)],
                        out_hbm.at[cid, pl.ds(lo + tb + 2 * ECHUNK, 64)])
        plsc.subcore_barrier()


@functools.lru_cache(maxsize=None)
def _sc_kernels():
    mesh = plsc.VectorSubcoreMesh(core_axis_name="c", subcore_axis_name="s",
                                  num_cores=NC, num_subcores=NS)
    deg = pl.kernel(
        _deg_kernel_body,
        out_type=jax.ShapeDtypeStruct((NC, NPAD), jnp.float32),
        mesh=mesh,
        scratch_types=[
            pltpu.VMEM((CHUNKS_PER_W, ECHUNK), jnp.int32),
            pltpu.VMEM((CHUNKS_PER_W, ECHUNK), jnp.float32),
            pltpu.VMEM((ECHUNK,), jnp.float32),
            pltpu.VMEM_SHARED((NPAD,), jnp.float32),
        ],
    )
    prop = pl.kernel(
        _prop_kernel_body,
        out_type=jax.ShapeDtypeStruct((NC, NPAD, D), jnp.float32),
        mesh=mesh,
        scratch_types=[
            pltpu.VMEM((C_MAX, ECHUNK), jnp.int32),
            pltpu.VMEM((C_MAX, ECHUNK), jnp.int32),
            pltpu.VMEM((C_MAX, ECHUNK), jnp.float32),
            pltpu.VMEM((ECHUNK,), jnp.int32),
            pltpu.VMEM((ECHUNK,), jnp.float32),
            pltpu.VMEM((ECHUNK, D), jnp.float32),
            pltpu.VMEM((ECHUNK, D), jnp.float32),
            pltpu.VMEM_SHARED((NHALF, D), jnp.float32),
            pltpu.SemaphoreType.DMA,
            pltpu.SemaphoreType.DMA,
        ],
    )
    return deg, prop


def _stage0_body(degp_ref, z_ref, dis_ref, y1_ref):
    degp = degp_ref[...]
    ones = jnp.ones((NC, 1), jnp.float32)
    deg = lax.dot_general(degp, ones, (((0,), (0,)), ((), ())),
                          preferred_element_type=jnp.float32) + 1.0
    dis = jnp.where(deg > 0, lax.rsqrt(deg), 0.0)
    dis_ref[...] = dis
    y1_ref[...] = dis * z_ref[...]


_stage0 = pl.pallas_call(
    _stage0_body,
    out_shape=(jax.ShapeDtypeStruct((NPAD, 1), jnp.float32),
               jax.ShapeDtypeStruct((NPAD, D), jnp.float32)),
)


def _stageL_body(accp_ref, y_ref, dis_ref, w_ref, b_ref, out_ref):
    dis = dis_ref[...]
    s = dis * (accp_ref[0] + accp_ref[1] + y_ref[...])
    h = lax.dot_general(s, w_ref[...], (((1,), (0,)), ((), ())),
                        preferred_element_type=jnp.float32) + b_ref[...]
    out_ref[...] = dis * jnp.maximum(h, 0.0)


_stageL = pl.pallas_call(
    _stageL_body,
    out_shape=jax.ShapeDtypeStruct((NPAD, D), jnp.float32),
)


def _stage3_body(accp_ref, y_ref, dis_ref, w_ref, b_ref, out_ref):
    s = dis_ref[...] * (accp_ref[0] + accp_ref[1] + y_ref[...])
    out_ref[...] = lax.dot_general(s, w_ref[...], (((1,), (0,)), ((), ())),
                                   preferred_element_type=jnp.float32) + b_ref[...]


_stage3 = pl.pallas_call(
    _stage3_body,
    out_shape=jax.ShapeDtypeStruct((NPAD, 128), jnp.float32),
)


def kernel(z, edge_index, edge_weight, W1, b1, W2, b2, W3, b3):
    src = edge_index[0].astype(jnp.int32)
    dst = edge_index[1].astype(jnp.int32)
    epad = EPAD - N_EDGES
    src_r = jnp.concatenate([src, jnp.zeros((epad,), jnp.int32)]).reshape(-1, ECHUNK)
    dst_r = jnp.concatenate([dst, jnp.zeros((epad,), jnp.int32)]).reshape(-1, ECHUNK)
    w_r = jnp.concatenate(
        [edge_weight, jnp.zeros((epad,), jnp.float32)]).reshape(-1, ECHUNK)
    z2d = jnp.zeros((NPAD, D), jnp.float32).at[:N_NODES, :2].set(z)

    W1p = jnp.zeros((D, D), jnp.float32).at[:2, :8].set(W1)
    b1p = jnp.zeros((1, D), jnp.float32).at[0, :8].set(b1)
    W2p = jnp.zeros((D, D), jnp.float32).at[:8, :CH].set(W2)
    b2p = jnp.zeros((1, D), jnp.float32).at[0, :CH].set(b2)
    W3p = jnp.zeros((D, 128), jnp.float32).at[:CH, :].set(W3)
    b3p = b3.reshape(1, 128)

    deg_kernel, prop_kernel = _sc_kernels()
    degp = deg_kernel(dst_r, w_r)
    dis, y1 = _stage0(degp, z2d)
    acc1 = prop_kernel(y1, src_r, dst_r, w_r)
    y2 = _stageL(acc1, y1, dis, W1p, b1p)
    acc2 = prop_kernel(y2, src_r, dst_r, w_r)
    y3 = _stageL(acc2, y2, dis, W2p, b2p)
    acc3 = prop_kernel(y3, src_r, dst_r, w_r)
    out = _stage3(acc3, y3, dis, W3p, b3p)
    return out[:N_NODES]
